# baseline (speedup 1.0000x reference)
import jax
import jax.numpy as jnp
from jax import lax
from jax.experimental import pallas as pl
from jax.experimental.pallas import tpu as pltpu
from jax.experimental.pallas import tpu_sc as plsc

N = 100000
G = 256
R = 200
H = 128
CD = 64
C = 14

BLK = 5000
NBLK_H = (N // 2) // BLK
T_ROWS = C * G
CH = 128
NW = 32
ROWS_PER_TILE = T_ROWS // 16

SPLIT = 390
NFULL1 = N // CH - SPLIT
TAIL = N - (N // CH) * CH


def _router_body(pos_ref, batch_ref, wl_ref, ci_ref, cidx_ref, cnt_ref):
    p = pos_ref[...]
    wl = wl_ref[...]
    logits = lax.dot_general(wl, p, (((1,), (1,)), ((), ())),
                             preferred_element_type=jnp.float32)
    m = jnp.max(logits, axis=0, keepdims=True)
    iota = lax.broadcasted_iota(jnp.int32, logits.shape, 0)
    idx = jnp.min(jnp.where(logits == m, iota, C), axis=0)
    b = batch_ref[0, 0, :]
    ci_ref[0, 0, :] = idx
    cidx_ref[0, 0, :] = idx * G + b
    seg = lax.broadcasted_iota(jnp.int32, (G, BLK), 0)
    eq = (batch_ref[0] == seg).astype(jnp.float32)
    part = jnp.sum(eq, axis=1, keepdims=True)

    @pl.when(pl.program_id(0) == 0)
    def _():
        cnt_ref[...] = part

    @pl.when(pl.program_id(0) != 0)
    def _():
        cnt_ref[...] += part


def _router(pos, batch3d, wl, off):
    return pl.pallas_call(
        _router_body,
        grid=(NBLK_H,),
        in_specs=[
            pl.BlockSpec((BLK, R), lambda i: (i + off, 0)),
            pl.BlockSpec((1, 1, BLK), lambda i: (i + off, 0, 0)),
            pl.BlockSpec((C, R), lambda i: (0, 0)),
        ],
        out_specs=[
            pl.BlockSpec((1, 1, BLK), lambda i: (i, 0, 0)),
            pl.BlockSpec((1, 1, BLK), lambda i: (i, 0, 0)),
            pl.BlockSpec((G, 1), lambda i: (0, 0)),
        ],
        out_shape=[
            jax.ShapeDtypeStruct((NBLK_H, 1, BLK), jnp.int32),
            jax.ShapeDtypeStruct((NBLK_H, 1, BLK), jnp.int32),
            jax.ShapeDtypeStruct((G, 1), jnp.float32),
        ],
    )(pos, batch3d, wl)


def _make_sc_body(t0, nfull, tail_rows):
    iters = (nfull + NW - 1) // NW

    def body(x_hbm, idx_hbm, zeros_hbm, out_hbm,
             idx_v, dat_v, idxt_v, datt_v, table):
        cid = lax.axis_index("c")
        sid = lax.axis_index("s")
        wid = sid * 2 + cid

        sl = pl.ds(sid * ROWS_PER_TILE, ROWS_PER_TILE)
        pltpu.sync_copy(zeros_hbm.at[sl], table.at[sl])
        plsc.subcore_barrier()

        def chunk(j, carry):
            tl = wid + NW * j

            @pl.when(tl < nfull)
            def _():
                pltpu.sync_copy(x_hbm.at[pl.ds((t0 + tl) * CH, CH)], dat_v)
                pltpu.sync_copy(idx_hbm.at[tl], idx_v)
                pltpu.sync_copy(dat_v, table.at[idx_v], add=True)

            return carry

        lax.fori_loop(0, iters, chunk, 0)

        if tail_rows:
            @pl.when(wid == NW - 1)
            def _():
                pltpu.sync_copy(
                    x_hbm.at[pl.ds((t0 + nfull) * CH, tail_rows)], datt_v)
                pltpu.sync_copy(idx_hbm.at[nfull, pl.ds(0, tail_rows)], idxt_v)
                pltpu.sync_copy(datt_v, table.at[idxt_v], add=True)

        plsc.subcore_barrier()
        pltpu.sync_copy(table.at[sl], out_hbm.at[cid, sl])

    return body


def _sc_scatter(x, idx_pad, zeros_tbl, t0, nfull, tail_rows):
    mesh = plsc.VectorSubcoreMesh(core_axis_name="c", subcore_axis_name="s")
    f = pl.kernel(
        _make_sc_body(t0, nfull, tail_rows),
        mesh=mesh,
        out_type=jax.ShapeDtypeStruct((2, T_ROWS, H), jnp.float32),
        scratch_types=[
            pltpu.VMEM((CH,), jnp.int32),
            pltpu.VMEM((CH, H), jnp.float32),
            pltpu.VMEM((max(TAIL, 8),), jnp.int32),
            pltpu.VMEM((max(TAIL, 8), H), jnp.float32),
            pltpu.VMEM_SHARED((T_ROWS, H), jnp.float32),
        ],
    )
    return f(x, idx_pad, zeros_tbl)


def _mlp_body(ta_ref, tb_ref, cnt_ref, w1_ref, b1_ref, g1_ref, bt1_ref,
              w2_ref, b2_ref, g2_ref, bt2_ref, out_ref):
    denom = jnp.maximum(cnt_ref[...], 1.0)
    halves = []
    for k in range(2):
        t = (ta_ref[0, pl.ds(k * G, G)] + ta_ref[1, pl.ds(k * G, G)]
             + tb_ref[0, pl.ds(k * G, G)] + tb_ref[1, pl.ds(k * G, G)])
        pooled = t / denom
        h = lax.dot_general(pooled, w1_ref[k], (((1,), (1,)), ((), ())),
                            preferred_element_type=jnp.float32) + b1_ref[k]
        m = jnp.mean(h, axis=0, keepdims=True)
        cen = h - m
        v = jnp.mean(cen * cen, axis=0, keepdims=True)
        h1 = g1_ref[k] * cen / jnp.sqrt(v + 1e-5) + bt1_ref[k]
        h1 = jnp.where(h1 >= 0, h1, 0.01 * h1)
        h2 = lax.dot_general(h1, w2_ref[k], (((1,), (1,)), ((), ())),
                             preferred_element_type=jnp.float32) + b2_ref[k]
        m2 = jnp.mean(h2, axis=0, keepdims=True)
        cen2 = h2 - m2
        v2 = jnp.mean(cen2 * cen2, axis=0, keepdims=True)
        h2n = g2_ref[k] * cen2 / jnp.sqrt(v2 + 1e-5) + bt2_ref[k]
        halves.append(jnp.where(h2n >= 0, h2n, 0.01 * h2n))
    out_ref[...] = jnp.concatenate(halves, axis=1)


def _mlp(ta, tb, counts, W1, b1, g1, bt1, W2, b2, g2, bt2):
    return pl.pallas_call(
        _mlp_body,
        grid=(C // 2,),
        in_specs=[
            pl.BlockSpec((2, 2 * G, H), lambda i: (0, i, 0)),
            pl.BlockSpec((2, 2 * G, H), lambda i: (0, i, 0)),
            pl.BlockSpec((G, 1), lambda i: (0, 0)),
            pl.BlockSpec((2, H, H), lambda i: (i, 0, 0)),
            pl.BlockSpec((2, 1, H), lambda i: (i, 0, 0)),
            pl.BlockSpec((2, 1, H), lambda i: (i, 0, 0)),
            pl.BlockSpec((2, 1, H), lambda i: (i, 0, 0)),
            pl.BlockSpec((2, CD, H), lambda i: (i, 0, 0)),
            pl.BlockSpec((2, 1, CD), lambda i: (i, 0, 0)),
            pl.BlockSpec((2, 1, CD), lambda i: (i, 0, 0)),
            pl.BlockSpec((2, 1, CD), lambda i: (i, 0, 0)),
        ],
        out_specs=pl.BlockSpec((G, 2 * CD), lambda i: (0, i)),
        out_shape=jax.ShapeDtypeStruct((G, C * CD), jnp.float32),
    )(ta, tb, counts, W1,
      b1.reshape(C, 1, H), g1.reshape(C, 1, H), bt1.reshape(C, 1, H),
      W2,
      b2.reshape(C, 1, CD), g2.reshape(C, 1, CD), bt2.reshape(C, 1, CD))


def kernel(x, batch, pos, size, Wl, W1, b1, g1, bt1, W2, b2, g2, bt2):
    batch32 = batch.astype(jnp.int32)
    batch3d = batch32.reshape(2 * NBLK_H, 1, BLK)
    zeros_tbl = jnp.zeros((T_ROWS, H), jnp.float32)

    ci0, cidx0, counts0 = _router(pos, batch3d, Wl, 0)
    cidx0f = cidx0.reshape(N // 2)
    idx_pad0 = cidx0f[:SPLIT * CH].reshape(SPLIT, CH)
    tables0 = _sc_scatter(x, idx_pad0, zeros_tbl, 0, SPLIT, 0)

    ci1, cidx1, counts1 = _router(pos, batch3d, Wl, NBLK_H)
    cidx1f = cidx1.reshape(N // 2)
    pad = (NFULL1 + 1) * CH - (N - SPLIT * CH)
    idx_pad1 = jnp.concatenate(
        [cidx0f[SPLIT * CH:], cidx1f, jnp.zeros((pad,), jnp.int32)]
    ).reshape(NFULL1 + 1, CH)
    tables1 = _sc_scatter(x, idx_pad1, zeros_tbl, SPLIT, NFULL1, TAIL)

    counts = counts0 + counts1
    out = _mlp(tables0, tables1, counts,
               W1, b1, g1, bt1, W2, b2, g2, bt2)
    ci = jnp.concatenate([ci0.reshape(N // 2), ci1.reshape(N // 2)])
    return (out, ci)


# speedup vs baseline: 2.4520x; 2.4520x over previous
import jax
import jax.numpy as jnp
from jax import lax
from jax.experimental import pallas as pl
from jax.experimental.pallas import tpu as pltpu
from jax.experimental.pallas import tpu_sc as plsc

N = 100000
G = 256
R = 200
H = 128
CD = 64
C = 14

CBLK = 5120
NBLK_H = 10
HALF0 = NBLK_H * CBLK
T_ROWS = C * G
SAC = T_ROWS
T_ROWS_P = T_ROWS + 128
CH = 128
NW = 32
ROWS_PER_TILE = T_ROWS_P // 16

SPLIT = HALF0 // CH
NFULL1 = (N - HALF0) // CH
TAIL = N - HALF0 - NFULL1 * CH
ITERS0 = 13
ITERS1 = 12


def _make_router_body(ncols):
    def body(posT_ref, batch_ref, wl_ref, ci_ref, cidx_ref, cnt_ref):
        p = posT_ref[...]
        wl = wl_ref[...]
        logits = lax.dot_general(wl, p, (((1,), (0,)), ((), ())),
                                 preferred_element_type=jnp.float32)
        m = jnp.max(logits, axis=0, keepdims=True)
        iota = lax.broadcasted_iota(jnp.int32, logits.shape, 0)
        idx = jnp.min(jnp.where(logits == m, iota, C), axis=0)
        b = batch_ref[...]
        ci_ref[...] = idx
        cidx_ref[...] = idx * G + b
        col = (pl.program_id(0) * CBLK
               + lax.broadcasted_iota(jnp.int32, (1, CBLK), 1))
        valid = col < ncols
        seg = lax.broadcasted_iota(jnp.int32, (G, CBLK), 0)
        eq = ((b[None, :] == seg) & valid).astype(jnp.float32)
        part = jnp.sum(eq, axis=1, keepdims=True)

        @pl.when(pl.program_id(0) == 0)
        def _():
            cnt_ref[...] = part

        @pl.when(pl.program_id(0) != 0)
        def _():
            cnt_ref[...] += part

    return body


def _router(posT, batch2d, wl, off, ncols):
    return pl.pallas_call(
        _make_router_body(ncols),
        grid=(NBLK_H,),
        in_specs=[
            pl.BlockSpec((R, CBLK), lambda i: (0, i + off)),
            pl.BlockSpec((CBLK,), lambda i: (i + off,)),
            pl.BlockSpec((C, R), lambda i: (0, 0)),
        ],
        out_specs=[
            pl.BlockSpec((CBLK,), lambda i: (i,)),
            pl.BlockSpec((CBLK,), lambda i: (i,)),
            pl.BlockSpec((G, 1), lambda i: (0, 0)),
        ],
        out_shape=[
            jax.ShapeDtypeStruct((ncols,), jnp.int32),
            jax.ShapeDtypeStruct((ncols,), jnp.int32),
            jax.ShapeDtypeStruct((G, 1), jnp.float32),
        ],
    )(posT, batch2d, wl)


def _make_sc_body(row_base, nfull, tail_tl, niters):
    def body(x_hbm, idx_hbm, out_hbm,
             idx_v, dat_v, sd0, sd1, sd2, si0, si1, si2, table):
        cid = lax.axis_index("c")
        sid = lax.axis_index("s")
        wid = sid * 2 + cid
        sems = ((sd0, si0), (sd1, si1), (sd2, si2))

        z = jnp.zeros((16,), jnp.float32)

        def zrow(r, carry):
            for k in range(H // 16):
                dat_v[0, r, pl.ds(k * 16, 16)] = z
            return carry

        lax.fori_loop(0, CH, zrow, 0)
        base = sid * ROWS_PER_TILE
        sl = pl.ds(base, ROWS_PER_TILE)
        pltpu.sync_copy(dat_v.at[0], table.at[pl.ds(base, CH)])
        pltpu.sync_copy(dat_v.at[0, pl.ds(0, ROWS_PER_TILE - CH)],
                        table.at[pl.ds(base + CH, ROWS_PER_TILE - CH)])
        plsc.subcore_barrier()

        def row0(tl):
            r = jnp.where(tl < nfull, row_base + tl * CH, 0)
            if tail_tl is not None:
                r = jnp.where(tl == tail_tl, row_base + tail_tl * CH - (CH - TAIL), r)
            return r

        def start(j, b):
            tl = wid + NW * j
            return (pltpu.async_copy(x_hbm.at[pl.ds(row0(tl), CH)],
                                     dat_v.at[b], sems[b][0]),
                    pltpu.async_copy(idx_hbm.at[tl], idx_v.at[b],
                                     sems[b][1]))

        pend = {}
        for j in range(min(3, niters)):
            pend[j] = start(j, j % 3)
        for j in range(niters):
            b = j % 3
            hd, hi = pend.pop(j)
            hd.wait()
            hi.wait()
            pltpu.sync_copy(dat_v.at[b], table.at[idx_v.at[b]], add=True)
            if j + 3 < niters:
                pend[j + 3] = start(j + 3, b)

        plsc.subcore_barrier()
        pltpu.sync_copy(table.at[sl], out_hbm.at[cid, sl])

    return body


def _sc_scatter(x, idx_pad, row_base, nfull, tail_tl, niters):
    mesh = plsc.VectorSubcoreMesh(core_axis_name="c", subcore_axis_name="s")
    f = pl.kernel(
        _make_sc_body(row_base, nfull, tail_tl, niters),
        mesh=mesh,
        out_type=jax.ShapeDtypeStruct((2, T_ROWS_P, H), jnp.float32),
        scratch_types=[
            pltpu.VMEM((3, CH), jnp.int32),
            pltpu.VMEM((3, CH, H), jnp.float32),
            pltpu.SemaphoreType.DMA,
            pltpu.SemaphoreType.DMA,
            pltpu.SemaphoreType.DMA,
            pltpu.SemaphoreType.DMA,
            pltpu.SemaphoreType.DMA,
            pltpu.SemaphoreType.DMA,
            pltpu.VMEM_SHARED((T_ROWS_P, H), jnp.float32),
        ],
    )
    return f(x, idx_pad)


def _mlp_body(ta_ref, tb_ref, cnt_ref, w1_ref, b1_ref, g1_ref, bt1_ref,
              w2_ref, b2_ref, g2_ref, bt2_ref, out_ref):
    denom = jnp.maximum(cnt_ref[...], 1.0)
    halves = []
    for k in range(2):
        t = (ta_ref[0, pl.ds(k * G, G)] + ta_ref[1, pl.ds(k * G, G)]
             + tb_ref[0, pl.ds(k * G, G)] + tb_ref[1, pl.ds(k * G, G)])
        pooled = t / denom
        h = lax.dot_general(pooled, w1_ref[k], (((1,), (1,)), ((), ())),
                            preferred_element_type=jnp.float32) + b1_ref[k]
        m = jnp.mean(h, axis=0, keepdims=True)
        cen = h - m
        v = jnp.mean(cen * cen, axis=0, keepdims=True)
        h1 = g1_ref[k] * cen / jnp.sqrt(v + 1e-5) + bt1_ref[k]
        h1 = jnp.where(h1 >= 0, h1, 0.01 * h1)
        h2 = lax.dot_general(h1, w2_ref[k], (((1,), (1,)), ((), ())),
                             preferred_element_type=jnp.float32) + b2_ref[k]
        m2 = jnp.mean(h2, axis=0, keepdims=True)
        cen2 = h2 - m2
        v2 = jnp.mean(cen2 * cen2, axis=0, keepdims=True)
        h2n = g2_ref[k] * cen2 / jnp.sqrt(v2 + 1e-5) + bt2_ref[k]
        halves.append(jnp.where(h2n >= 0, h2n, 0.01 * h2n))
    out_ref[...] = jnp.concatenate(halves, axis=1)


def _mlp(ta, tb, counts, W1, b1, g1, bt1, W2, b2, g2, bt2):
    return pl.pallas_call(
        _mlp_body,
        grid=(C // 2,),
        in_specs=[
            pl.BlockSpec((2, 2 * G, H), lambda i: (0, i, 0)),
            pl.BlockSpec((2, 2 * G, H), lambda i: (0, i, 0)),
            pl.BlockSpec((G, 1), lambda i: (0, 0)),
            pl.BlockSpec((2, H, H), lambda i: (i, 0, 0)),
            pl.BlockSpec((2, 1, H), lambda i: (i, 0, 0)),
            pl.BlockSpec((2, 1, H), lambda i: (i, 0, 0)),
            pl.BlockSpec((2, 1, H), lambda i: (i, 0, 0)),
            pl.BlockSpec((2, CD, H), lambda i: (i, 0, 0)),
            pl.BlockSpec((2, 1, CD), lambda i: (i, 0, 0)),
            pl.BlockSpec((2, 1, CD), lambda i: (i, 0, 0)),
            pl.BlockSpec((2, 1, CD), lambda i: (i, 0, 0)),
        ],
        out_specs=pl.BlockSpec((G, 2 * CD), lambda i: (0, i)),
        out_shape=jax.ShapeDtypeStruct((G, C * CD), jnp.float32),
    )(ta, tb, counts, W1,
      b1.reshape(C, 1, H), g1.reshape(C, 1, H), bt1.reshape(C, 1, H),
      W2,
      b2.reshape(C, 1, CD), g2.reshape(C, 1, CD), bt2.reshape(C, 1, CD))


def kernel(x, batch, pos, size, Wl, W1, b1, g1, bt1, W2, b2, g2, bt2):
    batch32 = batch.astype(jnp.int32)
    batch1d = batch32
    posT = pos.T
    zeros_tbl = jnp.zeros((T_ROWS, H), jnp.float32)

    ci0, cidx0, counts0 = _router(posT, batch1d, Wl, 0, HALF0)
    idx_pad0 = jnp.concatenate(
        [cidx0,
         jnp.full((ITERS0 * NW * CH - HALF0,), SAC, jnp.int32)]
    ).reshape(ITERS0 * NW, CH)
    tables0 = _sc_scatter(x, idx_pad0, 0, SPLIT, None, ITERS0)

    ci1, cidx1, counts1 = _router(posT, batch1d, Wl, NBLK_H, N - HALF0)
    cidx1f = cidx1
    idx_pad1 = jnp.concatenate(
        [cidx1f[:NFULL1 * CH],
         jnp.full((CH - TAIL,), SAC, jnp.int32),
         cidx1f[NFULL1 * CH:],
         jnp.full((ITERS1 * NW * CH - (NFULL1 + 1) * CH,), SAC, jnp.int32)]
    ).reshape(ITERS1 * NW, CH)
    tables1 = _sc_scatter(x, idx_pad1, HALF0, NFULL1,
                          NFULL1, ITERS1)

    counts = counts0 + counts1
    out = _mlp(tables0, tables1, counts,
               W1, b1, g1, bt1, W2, b2, g2, bt2)
    ci = jnp.concatenate([ci0, ci1])
    return (out, ci)
